# baseline (speedup 1.0000x reference)
import jax
import jax.numpy as jnp
from jax.experimental import pallas as pl
from jax.experimental.pallas import tpu as pltpu

_B = 10000


def _ltfgw_body(x_ref, tmpl_ref, feat_ref, q0_ref, alpha0_ref, out_ref,
                gmat_ref, bias_ref, oma_ref):
    @pl.when(pl.program_id(0) == 0)
    def _():
        alpha = jax.nn.sigmoid(alpha0_ref[0])
        q = jax.nn.softmax(q0_ref[...], axis=1)
        feats = feat_ref[...]
        g = jnp.sum(q[:, :, None] * feats, axis=1)
        c_ = jnp.sum(q * jnp.sum(feats * feats, axis=2), axis=1)
        tmpl = tmpl_ref[...]
        s = jnp.sum(q[:, :, None] * q[:, None, :] * (tmpl * tmpl),
                    axis=(1, 2))
        one_m_a = 1.0 - alpha
        gmat_ref[...] = (-2.0 * one_m_a) * g
        bias_ref[...] = (one_m_a * c_ + alpha * s)[None, :]
        oma_ref[0] = one_m_a

    xb = x_ref[...]
    x2 = jnp.sum(xb * xb, axis=1)
    dot = jax.lax.dot_general(
        xb, gmat_ref[...],
        dimension_numbers=(((1,), (1,)), ((), ())),
        preferred_element_type=jnp.float32,
    )
    out_ref[...] = oma_ref[0] * x2[:, None] + dot + bias_ref[...]


@jax.jit
def kernel(x, edge_index, templates, templates_features, q0, alpha0):
    del edge_index
    n, d = x.shape
    t, m, _ = templates.shape
    return pl.pallas_call(
        _ltfgw_body,
        grid=(n // _B,),
        in_specs=[
            pl.BlockSpec((_B, d), lambda i: (i, 0)),
            pl.BlockSpec((t, m, m), lambda i: (0, 0, 0)),
            pl.BlockSpec((t, m, d), lambda i: (0, 0, 0)),
            pl.BlockSpec((t, m), lambda i: (0, 0)),
            pl.BlockSpec(memory_space=pltpu.MemorySpace.SMEM),
        ],
        out_specs=pl.BlockSpec((_B, t), lambda i: (i, 0)),
        out_shape=jax.ShapeDtypeStruct((n, t), jnp.float32),
        scratch_shapes=[
            pltpu.VMEM((t, d), jnp.float32),
            pltpu.VMEM((1, t), jnp.float32),
            pltpu.SMEM((1,), jnp.float32),
        ],
    )(x, templates, templates_features, q0, alpha0)


# speedup vs baseline: 1.0865x; 1.0865x over previous
import jax
import jax.numpy as jnp
from jax.experimental import pallas as pl
from jax.experimental.pallas import tpu as pltpu

_B = 5000


def _ltfgw_body(x_ref, tmpl_ref, feat_ref, q0_ref, alpha0_ref, out_ref,
                gmat_ref, bias_ref, oma_ref):
    @pl.when(pl.program_id(0) == 0)
    def _():
        alpha = jax.nn.sigmoid(alpha0_ref[0])
        q = jax.nn.softmax(q0_ref[...], axis=1)
        feats = feat_ref[...]
        g = jnp.sum(q[:, :, None] * feats, axis=1)
        c_ = jnp.sum(q * jnp.sum(feats * feats, axis=2), axis=1)
        tmpl = tmpl_ref[...]
        s = jnp.sum(q[:, :, None] * q[:, None, :] * (tmpl * tmpl),
                    axis=(1, 2))
        one_m_a = 1.0 - alpha
        gmat_ref[...] = (-2.0 * one_m_a) * g
        bias_ref[...] = (one_m_a * c_ + alpha * s)[None, :]
        oma_ref[0] = one_m_a

    xb = x_ref[...]
    x2 = jnp.sum(xb * xb, axis=1)
    dot = jax.lax.dot_general(
        xb, gmat_ref[...],
        dimension_numbers=(((1,), (1,)), ((), ())),
        preferred_element_type=jnp.float32,
    )
    out_ref[...] = oma_ref[0] * x2[:, None] + dot + bias_ref[...]


@jax.jit
def kernel(x, edge_index, templates, templates_features, q0, alpha0):
    del edge_index
    n, d = x.shape
    t, m, _ = templates.shape
    return pl.pallas_call(
        _ltfgw_body,
        grid=(n // _B,),
        in_specs=[
            pl.BlockSpec((_B, d), lambda i: (i, 0)),
            pl.BlockSpec((t, m, m), lambda i: (0, 0, 0)),
            pl.BlockSpec((t, m, d), lambda i: (0, 0, 0)),
            pl.BlockSpec((t, m), lambda i: (0, 0)),
            pl.BlockSpec(memory_space=pltpu.MemorySpace.SMEM),
        ],
        out_specs=pl.BlockSpec((_B, t), lambda i: (i, 0)),
        out_shape=jax.ShapeDtypeStruct((n, t), jnp.float32),
        scratch_shapes=[
            pltpu.VMEM((t, d), jnp.float32),
            pltpu.VMEM((1, t), jnp.float32),
            pltpu.SMEM((1,), jnp.float32),
        ],
    )(x, templates, templates_features, q0, alpha0)
